# Initial kernel scaffold; baseline (speedup 1.0000x reference)
import jax
import jax.numpy as jnp
from jax.experimental import pallas as pl


def kernel(center, contexts, wordfreq, W_in, W_out):
    raise NotImplementedError("write your pallas kernel here")


# speedup vs baseline: 1.1369x; 1.1369x over previous
import functools

import jax
import jax.numpy as jnp
import numpy as np
from jax import lax
from jax.experimental import pallas as pl
from jax.experimental.pallas import tpu as pltpu
from jax.experimental.pallas import tpu_sc as plsc

NUM_NEGS = 20
_CHUNKS_PER_VOCAB = lambda v: (v + 1023) // 1024
_TINY = np.float32(np.finfo(np.float32).tiny)


_ROTS = (13, 15, 26, 6, 17, 29, 16, 24)


def _threefry_0_42(hi, lo):
    ks0 = jnp.uint32(0)
    ks1 = jnp.uint32(42)
    ks2 = jnp.uint32(0 ^ 42 ^ 0x1BD11BDA)
    ks = (ks0, ks1, ks2)
    x0 = hi + ks0
    x1 = lo + ks1
    for g in range(5):
        for r in _ROTS[(g % 2) * 4:(g % 2) * 4 + 4]:
            x0 = x0 + x1
            x1 = (x1 << r) | (x1 >> (32 - r))
            x1 = x0 ^ x1
        x0 = x0 + ks[(g + 1) % 3]
        x1 = x1 + ks[(g + 2) % 3] + jnp.uint32(g + 1)
    return x0 ^ x1


def _make_sample_body(n_chunks):
    def body(bhi_ref, blo_ref, lpad_ref, o_ref):
        j = pl.program_id(1)
        base_hi = bhi_ref[0, 0, j].astype(jnp.uint32)
        base_lo = blo_ref[0, 0, j].astype(jnp.uint32)
        off = (lax.broadcasted_iota(jnp.uint32, (8, 128), 0) * 128
               + lax.broadcasted_iota(jnp.uint32, (8, 128), 1))

        def one_chunk(c, smax, cbest):
            lo = base_lo + (off + jnp.uint32(c * 1024))
            carry_bit = (lo < base_lo).astype(jnp.uint32)
            hi = base_hi + carry_bit
            bits = _threefry_0_42(hi, lo)
            fb = (bits >> 9) | jnp.uint32(0x3F800000)
            u = lax.bitcast_convert_type(fb, jnp.float32) - jnp.float32(1.0)
            u = u + _TINY
            g = -jnp.log(-jnp.log(u))
            lch = lpad_ref[pl.ds(c * 8, 8), :]
            s = g + lch
            pred = s > smax
            return jnp.where(pred, s, smax), jnp.where(pred, c, cbest)

        smax = jnp.full((8, 128), -jnp.inf, jnp.float32)
        cbest = jnp.zeros((8, 128), jnp.int32)
        for c in range(n_chunks):
            smax, cbest = one_chunk(c, smax, cbest)

        m = jnp.max(smax)
        vidx = cbest * 1024 + off.astype(jnp.int32)
        win = jnp.min(jnp.where(smax == m, vidx, jnp.int32(2**30)))
        o_ref[0, 0, j] = win
    return body


def _sample_negative(wordfreq, rows):
    vocab = wordfreq.shape[0]
    n_chunks = _CHUNKS_PER_VOCAB(vocab)
    vpad = n_chunks * 1024
    logits = jnp.log(wordfreq.astype(jnp.float32))
    lpad = jnp.pad(logits, (0, vpad - vocab), constant_values=-1e9)
    lpad2 = lpad.reshape(n_chunks * 8, 128)

    outer = rows // 1024
    r = np.arange(rows, dtype=np.uint64) * np.uint64(vocab)
    bhi = (r >> np.uint64(32)).astype(np.int32).reshape(outer, 1, 1024)
    blo = (r & np.uint64(0xFFFFFFFF)).astype(np.uint32).view(np.int32)
    blo = blo.reshape(outer, 1, 1024)

    out = pl.pallas_call(
        _make_sample_body(n_chunks),
        grid=(outer, 1024),
        in_specs=[
            pl.BlockSpec((1, 1, 1024), lambda i, j: (i, 0, 0),
                         memory_space=pltpu.SMEM),
            pl.BlockSpec((1, 1, 1024), lambda i, j: (i, 0, 0),
                         memory_space=pltpu.SMEM),
            pl.BlockSpec((n_chunks * 8, 128), lambda i, j: (0, 0)),
        ],
        out_specs=pl.BlockSpec((1, 1, 1024), lambda i, j: (i, 0, 0),
                               memory_space=pltpu.SMEM),
        out_shape=jax.ShapeDtypeStruct((outer, 1, 1024), jnp.int32),
    )(jnp.asarray(bhi), jnp.asarray(blo), lpad2)
    return out.reshape(rows)


_NC, _NS = 2, 16
_NW = _NC * _NS


@functools.lru_cache(maxsize=None)
def _make_sc_gather(n_rows, dim):
    assert n_rows % (128 * _NW) == 0
    groups_per_w = n_rows // (128 * _NW)
    G = 1
    for cand in (6, 5, 4, 3, 2):
        if groups_per_w % cand == 0:
            G = cand
            break
    n_chunks = groups_per_w // G
    chunk_rows = G * 128

    mesh = plsc.VectorSubcoreMesh(core_axis_name="c", subcore_axis_name="s")

    @functools.partial(
        pl.kernel,
        mesh=mesh,
        compiler_params=pltpu.CompilerParams(use_tc_tiling_on_sc=False),
        out_type=jax.ShapeDtypeStruct((n_rows, dim), jnp.float32),
        scratch_types=[
            pltpu.VMEM((chunk_rows,), jnp.int32),
            pltpu.VMEM((chunk_rows, dim), jnp.float32),
            pltpu.SemaphoreType.DMA,
        ],
    )
    def gather_k(table_hbm, idx_hbm, out_hbm, idx_v, rows_v, sem):
        wid = lax.axis_index("s") * _NC + lax.axis_index("c")
        base_r = wid * groups_per_w * 128

        def chunk_body(t, _):
            r0 = base_r + t * chunk_rows
            pltpu.sync_copy(idx_hbm.at[pl.ds(r0, chunk_rows)], idx_v)
            copies = []
            for g in range(G):
                copies.append(pltpu.async_copy(
                    table_hbm.at[idx_v.at[pl.ds(g * 128, 128)]],
                    rows_v.at[pl.ds(g * 128, 128)], sem))
            for c in copies:
                c.wait()
            pltpu.sync_copy(rows_v, out_hbm.at[pl.ds(r0, chunk_rows)])
            return 0

        lax.fori_loop(0, n_chunks, chunk_body, 0, unroll=False)

    return gather_k


def _sc_gather(table, idx):
    n = idx.shape[0]
    k = _make_sc_gather(n, table.shape[1])
    return k(table, idx.astype(jnp.int32))


def kernel(center, contexts, wordfreq, W_in, W_out):
    B = center.shape[0]
    L = contexts.shape[1]
    rows = B * L * NUM_NEGS

    negative = _sample_negative(wordfreq, rows)

    centerV = _sc_gather(W_in, center.astype(jnp.int32))
    contextV = _sc_gather(W_out, contexts.reshape(-1).astype(jnp.int32))
    negativeV = _sc_gather(W_out, negative)

    return (centerV,
            contextV.reshape(B, L, W_out.shape[1]),
            negativeV.reshape(B, L * NUM_NEGS, W_out.shape[1]))
